# baseline (speedup 1.0000x reference)
import jax
import jax.numpy as jnp
from jax.experimental import pallas as pl
from jax.experimental.pallas import tpu as pltpu

_K = 64
_KP = 128
_D = 128
_BLK = 16384


def _vq_body(x_ref, emb2_ref, emb_ref, esq_ref, q_ref, loss_ref):
    i = pl.program_id(0)
    x = x_ref[...]
    emb2 = emb2_ref[...]
    emb = emb_ref[...]
    esq = esq_ref[...]
    scores2 = jax.lax.dot_general(x, emb2, (((1,), (1,)), ((), ())),
                                  preferred_element_type=jnp.float32)
    xsq = jnp.sum(x * x, axis=1, keepdims=True)
    dist = (xsq + esq) - scores2
    min_val = jnp.min(dist, axis=1, keepdims=True)
    iota = jax.lax.broadcasted_iota(jnp.int32, dist.shape, 1).astype(jnp.float32)
    masked = jnp.where(dist <= min_val, iota, float(_KP))
    idx = jnp.min(masked, axis=1, keepdims=True)
    onehot = jnp.where(iota == idx, 1.0, 0.0)
    q = jax.lax.dot_general(onehot, emb, (((1,), (0,)), ((), ())),
                            preferred_element_type=jnp.float32)
    q_ref[...] = q
    part = jnp.sum(min_val)

    @pl.when(i == 0)
    def _init():
        loss_ref[0, 0] = 0.0

    loss_ref[0, 0] += part


def kernel(inputs, embeddings):
    n, d = inputs.shape
    pad = jnp.zeros((_KP - _K, d), jnp.float32)
    emb_pad = jnp.concatenate([embeddings, pad], axis=0)
    esq = jnp.sum(embeddings ** 2, axis=1)
    esq_pad = jnp.concatenate(
        [esq, jnp.full((_KP - _K,), jnp.inf, jnp.float32)]).reshape(1, _KP)
    grid = (n // _BLK,)
    q, loss = pl.pallas_call(
        _vq_body,
        grid=grid,
        in_specs=[
            pl.BlockSpec((_BLK, d), lambda i: (i, 0)),
            pl.BlockSpec((_KP, d), lambda i: (0, 0)),
            pl.BlockSpec((_KP, d), lambda i: (0, 0)),
            pl.BlockSpec((1, _KP), lambda i: (0, 0)),
        ],
        out_specs=[
            pl.BlockSpec((_BLK, d), lambda i: (i, 0)),
            pl.BlockSpec(memory_space=pltpu.SMEM),
        ],
        out_shape=[
            jax.ShapeDtypeStruct((n, d), jnp.float32),
            jax.ShapeDtypeStruct((1, 1), jnp.float32),
        ],
    )(inputs, 2.0 * emb_pad, emb_pad, esq_pad)
    vq_loss = (2.0 / (n * d)) * loss[0, 0]
    return (q, vq_loss)


# speedup vs baseline: 1.0899x; 1.0899x over previous
import jax
import jax.numpy as jnp
from jax.experimental import pallas as pl
from jax.experimental.pallas import tpu as pltpu

_K = 64
_KP = 128
_D = 128
_BLK = 16384


def _vq_body(x_ref, emb2_ref, emb_ref, esq_ref, q_ref, loss_ref):
    i = pl.program_id(0)
    x = x_ref[...]
    emb2 = emb2_ref[...]
    emb = emb_ref[...]
    esq = esq_ref[...]
    scores2 = jax.lax.dot_general(x, emb2, (((1,), (1,)), ((), ())),
                                  preferred_element_type=jnp.float32)
    ones = jnp.ones((_KP, _D), jnp.float32)
    xsq = jax.lax.dot_general(x * x, ones, (((1,), (1,)), ((), ())),
                              preferred_element_type=jnp.float32)
    dist = (xsq + esq) - scores2
    min_val = jnp.min(dist, axis=1, keepdims=True)
    iota = jax.lax.broadcasted_iota(jnp.int32, dist.shape, 1).astype(jnp.float32)
    masked = jnp.where(dist <= min_val, iota, float(_KP))
    idx = jnp.min(masked, axis=1, keepdims=True)
    onehot = jnp.where(iota == idx, 1.0, 0.0)
    q = jax.lax.dot_general(onehot, emb, (((1,), (0,)), ((), ())),
                            preferred_element_type=jnp.float32)
    q_ref[...] = q
    part = jnp.sum(min_val)

    @pl.when(i == 0)
    def _init():
        loss_ref[0, 0] = 0.0

    loss_ref[0, 0] += part


def kernel(inputs, embeddings):
    n, d = inputs.shape
    pad = jnp.zeros((_KP - _K, d), jnp.float32)
    emb_pad = jnp.concatenate([embeddings, pad], axis=0)
    esq = jnp.sum(embeddings ** 2, axis=1)
    esq_pad = jnp.concatenate(
        [esq, jnp.full((_KP - _K,), jnp.inf, jnp.float32)]).reshape(1, _KP)
    grid = (n // _BLK,)
    q, loss = pl.pallas_call(
        _vq_body,
        grid=grid,
        in_specs=[
            pl.BlockSpec((_BLK, d), lambda i: (i, 0)),
            pl.BlockSpec((_KP, d), lambda i: (0, 0)),
            pl.BlockSpec((_KP, d), lambda i: (0, 0)),
            pl.BlockSpec((1, _KP), lambda i: (0, 0)),
        ],
        out_specs=[
            pl.BlockSpec((_BLK, d), lambda i: (i, 0)),
            pl.BlockSpec(memory_space=pltpu.SMEM),
        ],
        out_shape=[
            jax.ShapeDtypeStruct((n, d), jnp.float32),
            jax.ShapeDtypeStruct((1, 1), jnp.float32),
        ],
    )(inputs, 2.0 * emb_pad, emb_pad, esq_pad)
    vq_loss = (2.0 / (n * d)) * loss[0, 0]
    return (q, vq_loss)
